# Initial kernel scaffold; baseline (speedup 1.0000x reference)
import jax
import jax.numpy as jnp
from jax.experimental import pallas as pl


def kernel(X, params):
    raise NotImplementedError("write your pallas kernel here")


# speedup vs baseline: 91.2441x; 91.2441x over previous
import functools

import jax
import jax.numpy as jnp
from jax import lax
from jax.experimental import pallas as pl
from jax.experimental.pallas import tpu as pltpu
from jax.experimental.pallas import tpu_sc as plsc

B = 256
D = 2048
A = 8
O = 16
L = 16
NC = 2
NS = 16
NW = NC * NS
DW = D // NW
XC = 16
PC = 8
NCH = DW // XC
GROUPS = B // L

_NET = (
    (0, 1), (2, 3), (4, 5), (6, 7),
    (0, 2), (1, 3), (4, 6), (5, 7),
    (1, 2), (5, 6),
    (0, 4), (1, 5), (2, 6), (3, 7),
    (2, 4), (3, 5),
    (1, 2), (3, 4), (5, 6),
)

_mesh = plsc.VectorSubcoreMesh(core_axis_name="c", subcore_axis_name="s")


@functools.partial(
    pl.kernel,
    out_type=jax.ShapeDtypeStruct((B, D * O), jnp.float32),
    mesh=_mesh,
    scratch_types=[
        pltpu.VMEM((B, XC * A), jnp.float32),
        pltpu.VMEM((PC, 2 ** A * O), jnp.float32),
        pltpu.VMEM((B, PC * O), jnp.float32),
    ],
    compiler_params=pltpu.CompilerParams(needs_layout_passes=False),
)
def _hoa(x_hbm, p_hbm, out_hbm, x_v, p_v, o_v):
    wid = lax.axis_index("s") * NC + lax.axis_index("c")
    d0 = wid * DW
    iota = lax.iota(jnp.int32, L)

    def chunk_body(ci, carry):
        dc = d0 + ci * XC
        pltpu.sync_copy(x_hbm.at[:, pl.ds(dc * A, XC * A)], x_v)

        def half_body(h, carry):
            dp = dc + h * PC
            pltpu.sync_copy(p_hbm.at[pl.ds(dp, PC)], p_v)

            def d_body(ds_, carry):
                xcol = jnp.full((L,), (h * PC + ds_) * A, jnp.int32)
                ocol = jnp.full((L,), ds_ * O, jnp.int32)
                prow = jnp.full((L,), ds_, jnp.int32)

                def g_body(g, carry):
                    b_idx = g * L + iota
                    vs = [plsc.load_gather(x_v, [b_idx, xcol + k])
                          for k in range(A)]
                    ms = [jnp.full((L,), 1 << k, jnp.int32) for k in range(A)]
                    for (i, j) in _NET:
                        p = vs[i] <= vs[j]
                        lo = jnp.minimum(vs[i], vs[j])
                        hi = jnp.maximum(vs[i], vs[j])
                        ml = jnp.where(p, ms[i], ms[j])
                        mh = jnp.where(p, ms[j], ms[i])
                        vs[i], vs[j], ms[i], ms[j] = lo, hi, ml, mh
                    cs = [vs[0]] + [vs[k] - vs[k - 1] for k in range(1, A)]
                    ind = ms[A - 1]
                    rows = [None] * A
                    rows[A - 1] = ind << 4
                    for k in range(A - 2, -1, -1):
                        ind = ind + ms[k]
                        rows[k] = ind << 4
                    acc = [None] * O
                    for k in range(A):
                        for o in range(O):
                            g_ = plsc.load_gather(p_v, [prow, rows[k] + o])
                            if k == 0:
                                acc[o] = cs[0] * g_
                            else:
                                acc[o] = acc[o] + cs[k] * g_
                    for o in range(O):
                        plsc.store_scatter(o_v, [b_idx, ocol + o], acc[o])
                    return carry

                lax.fori_loop(0, GROUPS, g_body, 0)
                return carry

            lax.fori_loop(0, PC, d_body, 0)
            pltpu.sync_copy(o_v, out_hbm.at[:, pl.ds(dp * O, PC * O)])
            return carry

        lax.fori_loop(0, 2, half_body, 0)
        return carry

    lax.fori_loop(0, NCH, chunk_body, 0)


def kernel(X, params):
    out = _hoa(X.reshape(B, D * A), params.reshape(D, 2 ** A * O))
    return out.reshape(B, D, O)


# speedup vs baseline: 220.5770x; 2.4174x over previous
import functools

import jax
import jax.numpy as jnp
from jax import lax
from jax.experimental import pallas as pl
from jax.experimental.pallas import tpu as pltpu
from jax.experimental.pallas import tpu_sc as plsc

B = 256
D = 2048
A = 8
O = 16
R = 2 ** A
L = 16
NC = 2
NS = 16
NW = NC * NS
DW = D // NW
XC = 16
PC = 8
NCH = DW // XC
GROUPS = B // L

_NET = (
    (0, 1), (2, 3), (4, 5), (6, 7),
    (0, 2), (1, 3), (4, 6), (5, 7),
    (1, 2), (5, 6),
    (0, 4), (1, 5), (2, 6), (3, 7),
    (2, 4), (3, 5),
    (1, 2), (3, 4), (5, 6),
)

_mesh = plsc.VectorSubcoreMesh(core_axis_name="c", subcore_axis_name="s")


@functools.partial(
    pl.kernel,
    out_type=jax.ShapeDtypeStruct((D * O, B), jnp.float32),
    mesh=_mesh,
    scratch_types=[
        pltpu.VMEM((XC * A, B), jnp.float32),
        pltpu.VMEM((PC, O * R), jnp.float32),
        pltpu.VMEM((PC * O, B), jnp.float32),
    ],
    compiler_params=pltpu.CompilerParams(needs_layout_passes=False),
)
def _hoa(x_hbm, p_hbm, out_hbm, x_v, p_v, o_v):
    wid = lax.axis_index("s") * NC + lax.axis_index("c")
    d0 = wid * DW
    iota = lax.iota(jnp.int32, L)

    def chunk_body(ci, carry):
        dc = d0 + ci * XC
        pltpu.sync_copy(x_hbm.at[pl.ds(dc * A, XC * A)], x_v)

        def half_body(h, carry):
            dp = dc + h * PC
            pltpu.sync_copy(p_hbm.at[pl.ds(dp, PC)], p_v)

            def d_body(ds_, carry):
                c0 = (h * PC + ds_) * A
                orow0 = ds_ * O
                prow = jnp.full((L,), ds_, jnp.int32)

                def g_body(g, carry):
                    gb = g * L
                    vs = [x_v[c0 + k, pl.ds(gb, L)] for k in range(A)]
                    ms = [jnp.full((L,), 1 << k, jnp.int32) for k in range(A)]
                    for (i, j) in _NET:
                        p = vs[i] <= vs[j]
                        lo = jnp.minimum(vs[i], vs[j])
                        hi = jnp.maximum(vs[i], vs[j])
                        ml = jnp.where(p, ms[i], ms[j])
                        mh = jnp.where(p, ms[j], ms[i])
                        vs[i], vs[j], ms[i], ms[j] = lo, hi, ml, mh
                    cs = [vs[0]] + [vs[k] - vs[k - 1] for k in range(1, A)]
                    ind = ms[A - 1]
                    rows = [None] * A
                    rows[A - 1] = ind
                    for k in range(A - 2, -1, -1):
                        ind = ind + ms[k]
                        rows[k] = ind
                    acc = [None] * O
                    for k in range(A):
                        for o in range(O):
                            g_ = plsc.load_gather(p_v, [prow, rows[k] + o * R])
                            if k == 0:
                                acc[o] = cs[0] * g_
                            else:
                                acc[o] = acc[o] + cs[k] * g_
                    for o in range(O):
                        o_v[orow0 + o, pl.ds(gb, L)] = acc[o]
                    return carry

                lax.fori_loop(0, GROUPS, g_body, 0)
                return carry

            lax.fori_loop(0, PC, d_body, 0)
            pltpu.sync_copy(o_v, out_hbm.at[pl.ds(dp * O, PC * O)])
            return carry

        lax.fori_loop(0, 2, half_body, 0)
        return carry

    lax.fori_loop(0, NCH, chunk_body, 0)


def kernel(X, params):
    x_t = X.reshape(B, D * A).T
    p_t = params.transpose(0, 2, 1).reshape(D, O * R)
    out_t = _hoa(x_t, p_t)
    return out_t.reshape(D, O, B).transpose(2, 0, 1)


# speedup vs baseline: 222.4559x; 1.0085x over previous
import functools

import jax
import jax.numpy as jnp
from jax import lax
from jax.experimental import pallas as pl
from jax.experimental.pallas import tpu as pltpu
from jax.experimental.pallas import tpu_sc as plsc

B = 256
D = 2048
A = 8
O = 16
R = 2 ** A
L = 16
NC = 2
NS = 16
NW = NC * NS
DW = D // NW
XC = 16
PC = 8
NCH = DW // XC
GROUPS = B // L

_NET = (
    (0, 1), (2, 3), (4, 5), (6, 7),
    (0, 2), (1, 3), (4, 6), (5, 7),
    (1, 2), (5, 6),
    (0, 4), (1, 5), (2, 6), (3, 7),
    (2, 4), (3, 5),
    (1, 2), (3, 4), (5, 6),
)

_mesh = plsc.VectorSubcoreMesh(core_axis_name="c", subcore_axis_name="s")


@functools.partial(
    pl.kernel,
    out_type=jax.ShapeDtypeStruct((D * O, B), jnp.float32),
    mesh=_mesh,
    scratch_types=[
        pltpu.VMEM((XC * A, B), jnp.float32),
        pltpu.VMEM((PC, O * R), jnp.float32),
        pltpu.VMEM((PC * O, B), jnp.float32),
        pltpu.VMEM((O, L), jnp.float32),
    ],
    compiler_params=pltpu.CompilerParams(needs_layout_passes=False),
)
def _hoa(x_hbm, p_hbm, out_hbm, x_v, p_v, o_v, r255_v):
    wid = lax.axis_index("s") * NC + lax.axis_index("c")
    d0 = wid * DW
    iota = lax.iota(jnp.int32, L)

    def chunk_body(ci, carry):
        dc = d0 + ci * XC
        pltpu.sync_copy(x_hbm.at[pl.ds(dc * A, XC * A)], x_v)

        def half_body(h, carry):
            dp = dc + h * PC
            pltpu.sync_copy(p_hbm.at[pl.ds(dp, PC)], p_v)

            def d_body(ds_, carry):
                c0 = (h * PC + ds_) * A
                orow0 = ds_ * O
                prow = jnp.full((L,), ds_, jnp.int32)
                for o in range(O):
                    v16 = p_v[ds_, pl.ds(o * R + 240, L)]
                    r255_v[o, :] = jnp.full((L,), v16[15], jnp.float32)

                def g_body(g, carry):
                    gb = g * L
                    vs = [x_v[c0 + k, pl.ds(gb, L)] for k in range(A)]
                    ms = [jnp.full((L,), 1 << k, jnp.int32) for k in range(A)]
                    for (i, j) in _NET:
                        p = vs[i] <= vs[j]
                        lo = jnp.minimum(vs[i], vs[j])
                        hi = jnp.maximum(vs[i], vs[j])
                        ml = jnp.where(p, ms[i], ms[j])
                        mh = jnp.where(p, ms[j], ms[i])
                        vs[i], vs[j], ms[i], ms[j] = lo, hi, ml, mh
                    cs = [vs[0]] + [vs[k] - vs[k - 1] for k in range(1, A)]
                    ind = ms[A - 1]
                    rows = [None] * A
                    rows[A - 1] = ind
                    for k in range(A - 2, -1, -1):
                        ind = ind + ms[k]
                        rows[k] = ind
                    acc = [cs[0] * r255_v[o, :] for o in range(O)]
                    for k in range(1, A):
                        for o in range(O):
                            g_ = plsc.load_gather(p_v, [prow, rows[k] + o * R])
                            acc[o] = acc[o] + cs[k] * g_
                    for o in range(O):
                        o_v[orow0 + o, pl.ds(gb, L)] = acc[o]
                    return carry

                lax.fori_loop(0, GROUPS, g_body, 0)
                return carry

            lax.fori_loop(0, PC, d_body, 0)
            pltpu.sync_copy(o_v, out_hbm.at[pl.ds(dp * O, PC * O)])
            return carry

        lax.fori_loop(0, 2, half_body, 0)
        return carry

    lax.fori_loop(0, NCH, chunk_body, 0)


def kernel(X, params):
    x_t = X.reshape(B, D * A).T
    p_t = params.transpose(0, 2, 1).reshape(D, O * R)
    out_t = _hoa(x_t, p_t)
    return out_t.reshape(D, O, B).transpose(2, 0, 1)
